# Initial kernel scaffold; baseline (speedup 1.0000x reference)
import jax
import jax.numpy as jnp
from jax.experimental import pallas as pl


def kernel(x, batch, W, b):
    raise NotImplementedError("write your pallas kernel here")


# speedup vs baseline: 4.4757x; 4.4757x over previous
import functools

import jax
import jax.numpy as jnp
from jax import lax
from jax.experimental import pallas as pl
from jax.experimental.pallas import tpu as pltpu
from jax.experimental.pallas import tpu_sc as plsc

N_NODES = 100000
D = 128
G = 1000

NC = 2
NS = 16
NW = NC * NS

C = 80
NCHUNKS = N_NODES // C
TRIPS = -(-NCHUNKS // NW)

CNT_W = 16
RPS = 64
RPS_LAST = G - (NS - 1) * RPS


def _sc_segment_sums(x, batch, ones_c, zrow_c, zcnt_c):
    mesh = plsc.VectorSubcoreMesh(core_axis_name="c", subcore_axis_name="s")

    @functools.partial(
        pl.kernel,
        out_type=[
            jax.ShapeDtypeStruct((NC, G, D), jnp.float32),
            jax.ShapeDtypeStruct((NC, G, CNT_W), jnp.float32),
        ],
        mesh=mesh,
        compiler_params=pltpu.CompilerParams(use_tc_tiling_on_sc=False),
        scratch_types=[
            pltpu.VMEM((C, D), jnp.float32),
            pltpu.VMEM((C,), jnp.int32),
            pltpu.VMEM((C, CNT_W), jnp.float32),
            pltpu.VMEM_SHARED((G, D), jnp.float32),
            pltpu.VMEM_SHARED((G, CNT_W), jnp.float32),
        ],
    )
    def k(x_hbm, b_hbm, ones_hbm, zrow_hbm, zcnt_hbm, sums_hbm, cnts_hbm,
          xbuf, idx, ones, acc, accc):
        cid = lax.axis_index("c")
        sid = lax.axis_index("s")
        w = cid * NS + sid

        pltpu.sync_copy(ones_hbm, ones)

        r0 = sid * RPS

        @pl.when(sid < NS - 1)
        def _():
            pltpu.sync_copy(zrow_hbm, acc.at[pl.ds(r0, RPS)])
            pltpu.sync_copy(zcnt_hbm, accc.at[pl.ds(r0, RPS)])

        @pl.when(sid == NS - 1)
        def _():
            pltpu.sync_copy(zrow_hbm.at[pl.ds(0, RPS_LAST)],
                            acc.at[pl.ds((NS - 1) * RPS, RPS_LAST)])
            pltpu.sync_copy(zcnt_hbm.at[pl.ds(0, RPS_LAST)],
                            accc.at[pl.ds((NS - 1) * RPS, RPS_LAST)])

        plsc.subcore_barrier()

        def body(i, carry):
            chunk = w + i * NW

            @pl.when(chunk < NCHUNKS)
            def _():
                off = chunk * C
                pltpu.sync_copy(b_hbm.at[pl.ds(off, C)], idx)
                pltpu.sync_copy(x_hbm.at[pl.ds(off, C), :], xbuf)
                pltpu.sync_copy(xbuf, acc.at[idx], add=True)
                pltpu.sync_copy(ones, accc.at[idx], add=True)

            return carry

        lax.fori_loop(0, TRIPS, body, 0)

        plsc.subcore_barrier()

        @pl.when(sid < NS - 1)
        def _():
            pltpu.sync_copy(acc.at[pl.ds(r0, RPS)],
                            sums_hbm.at[cid, pl.ds(r0, RPS), :])
            pltpu.sync_copy(accc.at[pl.ds(r0, RPS)],
                            cnts_hbm.at[cid, pl.ds(r0, RPS), :])

        @pl.when(sid == NS - 1)
        def _():
            pltpu.sync_copy(acc.at[pl.ds((NS - 1) * RPS, RPS_LAST)],
                            sums_hbm.at[cid, pl.ds((NS - 1) * RPS, RPS_LAST), :])
            pltpu.sync_copy(accc.at[pl.ds((NS - 1) * RPS, RPS_LAST)],
                            cnts_hbm.at[cid, pl.ds((NS - 1) * RPS, RPS_LAST), :])

    return k(x, batch, ones_c, zrow_c, zcnt_c)


def _tc_finish(psums, pcnts, W, b2d):
    def body(ps_ref, pc_ref, w_ref, b_ref, o_ref):
        sums = ps_ref[0] + ps_ref[1]
        cnt = pc_ref[0, :, 0:1] + pc_ref[1, :, 0:1]
        pooled = sums / jnp.maximum(cnt, 1.0)
        o_ref[...] = (
            jnp.dot(pooled, w_ref[...], preferred_element_type=jnp.float32)
            + b_ref[...]
        )

    return pl.pallas_call(
        body,
        out_shape=jax.ShapeDtypeStruct((G, D), jnp.float32),
    )(psums, pcnts, W, b2d)


def kernel(x, batch, W, b):
    ones_c = jnp.ones((C, CNT_W), dtype=jnp.float32)
    zrow_c = jnp.zeros((RPS, D), dtype=jnp.float32)
    zcnt_c = jnp.zeros((RPS, CNT_W), dtype=jnp.float32)
    psums, pcnts = _sc_segment_sums(x, batch, ones_c, zrow_c, zcnt_c)
    return _tc_finish(psums, pcnts, W, b.reshape(1, D))


# speedup vs baseline: 6.7062x; 1.4984x over previous
import functools

import jax
import jax.numpy as jnp
from jax import lax
from jax.experimental import pallas as pl
from jax.experimental.pallas import tpu as pltpu
from jax.experimental.pallas import tpu_sc as plsc

N_NODES = 100000
D = 128
G = 1000

NC = 2
NS = 16
NW = NC * NS

C = 80
NCHUNKS = N_NODES // C
BIG = 5
ROWS = BIG * C
NBIG = N_NODES // ROWS
TRIPS = -(-NBIG // NW)

CNT_W = 16
RPS = 64
RPS_LAST = G - (NS - 1) * RPS


def _sc_segment_sums(x, batch2d, ones_c, zrow_c, zcnt_c):
    mesh = plsc.VectorSubcoreMesh(core_axis_name="c", subcore_axis_name="s")

    @functools.partial(
        pl.kernel,
        out_type=[
            jax.ShapeDtypeStruct((NC, G, D), jnp.float32),
            jax.ShapeDtypeStruct((NC, G, CNT_W), jnp.float32),
        ],
        mesh=mesh,
        compiler_params=pltpu.CompilerParams(use_tc_tiling_on_sc=False),
        scratch_types=[
            pltpu.VMEM((ROWS, D), jnp.float32),
            pltpu.VMEM((ROWS, D), jnp.float32),
            pltpu.VMEM((BIG, C), jnp.int32),
            pltpu.VMEM((BIG, C), jnp.int32),
            pltpu.VMEM((C, CNT_W), jnp.float32),
            pltpu.VMEM_SHARED((G, D), jnp.float32),
            pltpu.VMEM_SHARED((G, CNT_W), jnp.float32),
            pltpu.SemaphoreType.DMA,
            pltpu.SemaphoreType.DMA,
            pltpu.SemaphoreType.DMA,
            pltpu.SemaphoreType.DMA,
            pltpu.SemaphoreType.DMA,
            pltpu.SemaphoreType.DMA,
        ],
    )
    def k(x_hbm, b_hbm, ones_hbm, zrow_hbm, zcnt_hbm, sums_hbm, cnts_hbm,
          xb0, xb1, ib0, ib1, ones, acc, accc,
          slx0, slx1, sli0, sli1, ssc0, ssc1):
        cid = lax.axis_index("c")
        sid = lax.axis_index("s")
        w = cid * NS + sid

        xbufs = (xb0, xb1)
        ibufs = (ib0, ib1)
        slx = (slx0, slx1)
        sli = (sli0, sli1)
        ssc = (ssc0, ssc1)

        def start_load(i):
            p = i % 2
            big = w + i * NW

            @pl.when(big < NBIG)
            def _():
                pltpu.async_copy(x_hbm.at[pl.ds(big * ROWS, ROWS), :],
                                 xbufs[p], slx[p])
                pltpu.async_copy(b_hbm.at[pl.ds(big * BIG, BIG), :],
                                 ibufs[p], sli[p])

        def wait_load(i):
            p = i % 2
            big = w + i * NW

            @pl.when(big < NBIG)
            def _():
                pltpu.make_async_copy(x_hbm.at[pl.ds(0, ROWS), :],
                                      xbufs[p], slx[p]).wait()
                pltpu.make_async_copy(b_hbm.at[pl.ds(0, BIG), :],
                                      ibufs[p], sli[p]).wait()

        def start_scat(i):
            p = i % 2
            big = w + i * NW

            @pl.when(big < NBIG)
            def _():
                for j in range(BIG):
                    pltpu.async_copy(xbufs[p].at[pl.ds(j * C, C)],
                                     acc.at[ibufs[p].at[j]], ssc[p], add=True)
                    pltpu.async_copy(ones, accc.at[ibufs[p].at[j]],
                                     ssc[p], add=True)

        def wait_scat(i):
            p = i % 2
            big = w + i * NW

            @pl.when(big < NBIG)
            def _():
                for j in range(BIG):
                    pltpu.make_async_copy(xbufs[p].at[pl.ds(j * C, C)],
                                          acc.at[ibufs[p].at[j]], ssc[p]).wait()
                    pltpu.make_async_copy(ones, accc.at[ibufs[p].at[j]],
                                          ssc[p]).wait()

        start_load(0)
        start_load(1)
        pltpu.sync_copy(ones_hbm, ones)

        r0 = sid * RPS

        @pl.when(sid < NS - 1)
        def _():
            pltpu.sync_copy(zrow_hbm, acc.at[pl.ds(r0, RPS)])
            pltpu.sync_copy(zcnt_hbm, accc.at[pl.ds(r0, RPS)])

        @pl.when(sid == NS - 1)
        def _():
            pltpu.sync_copy(zrow_hbm.at[pl.ds(0, RPS_LAST)],
                            acc.at[pl.ds((NS - 1) * RPS, RPS_LAST)])
            pltpu.sync_copy(zcnt_hbm.at[pl.ds(0, RPS_LAST)],
                            accc.at[pl.ds((NS - 1) * RPS, RPS_LAST)])

        plsc.subcore_barrier()

        for i in range(TRIPS):
            wait_load(i)
            start_scat(i)
            if i >= 1:
                wait_scat(i - 1)
                if i + 1 < TRIPS:
                    start_load(i + 1)
        wait_scat(TRIPS - 1)

        plsc.subcore_barrier()

        @pl.when(sid < NS - 1)
        def _():
            pltpu.sync_copy(acc.at[pl.ds(r0, RPS)],
                            sums_hbm.at[cid, pl.ds(r0, RPS), :])
            pltpu.sync_copy(accc.at[pl.ds(r0, RPS)],
                            cnts_hbm.at[cid, pl.ds(r0, RPS), :])

        @pl.when(sid == NS - 1)
        def _():
            pltpu.sync_copy(acc.at[pl.ds((NS - 1) * RPS, RPS_LAST)],
                            sums_hbm.at[cid, pl.ds((NS - 1) * RPS, RPS_LAST), :])
            pltpu.sync_copy(accc.at[pl.ds((NS - 1) * RPS, RPS_LAST)],
                            cnts_hbm.at[cid, pl.ds((NS - 1) * RPS, RPS_LAST), :])

    return k(x, batch2d, ones_c, zrow_c, zcnt_c)


def _tc_finish(psums, pcnts, W, b2d):
    def body(ps_ref, pc_ref, w_ref, b_ref, o_ref):
        sums = ps_ref[0] + ps_ref[1]
        cnt = pc_ref[0, :, 0:1] + pc_ref[1, :, 0:1]
        pooled = sums / jnp.maximum(cnt, 1.0)
        o_ref[...] = (
            jnp.dot(pooled, w_ref[...], preferred_element_type=jnp.float32)
            + b_ref[...]
        )

    return pl.pallas_call(
        body,
        out_shape=jax.ShapeDtypeStruct((G, D), jnp.float32),
    )(psums, pcnts, W, b2d)


def kernel(x, batch, W, b):
    ones_c = jnp.ones((C, CNT_W), dtype=jnp.float32)
    zrow_c = jnp.zeros((RPS, D), dtype=jnp.float32)
    zcnt_c = jnp.zeros((RPS, CNT_W), dtype=jnp.float32)
    psums, pcnts = _sc_segment_sums(x, batch.reshape(NCHUNKS, C),
                                    ones_c, zrow_c, zcnt_c)
    return _tc_finish(psums, pcnts, W, b.reshape(1, D))


# speedup vs baseline: 6.7837x; 1.0116x over previous
import functools

import jax
import jax.numpy as jnp
from jax import lax
from jax.experimental import pallas as pl
from jax.experimental.pallas import tpu as pltpu
from jax.experimental.pallas import tpu_sc as plsc

N_NODES = 100000
D = 128
G = 1000

NC = 2
NS = 16
NW = NC * NS

C = 80
NCHUNKS = N_NODES // C
BIG = 5
ROWS = BIG * C
NBIG = N_NODES // ROWS
TRIPS = -(-NBIG // NW)

CNT_W = 16
RPS = 64
RPS_LAST = G - (NS - 1) * RPS


def _sc_segment_sums(x, batch2d, ones_c, zrow_c, zcnt_c):
    mesh = plsc.VectorSubcoreMesh(core_axis_name="c", subcore_axis_name="s")

    @functools.partial(
        pl.kernel,
        out_type=[
            jax.ShapeDtypeStruct((NC, G, D), jnp.float32),
            jax.ShapeDtypeStruct((NC, G, CNT_W), jnp.float32),
        ],
        mesh=mesh,
        compiler_params=pltpu.CompilerParams(use_tc_tiling_on_sc=False),
        scratch_types=[
            pltpu.VMEM((ROWS, D), jnp.float32),
            pltpu.VMEM((ROWS, D), jnp.float32),
            pltpu.VMEM((BIG, C), jnp.int32),
            pltpu.VMEM((BIG, C), jnp.int32),
            pltpu.VMEM((C, CNT_W), jnp.float32),
            pltpu.VMEM_SHARED((G, D), jnp.float32),
            pltpu.VMEM_SHARED((G, CNT_W), jnp.float32),
            pltpu.SemaphoreType.DMA,
            pltpu.SemaphoreType.DMA,
            pltpu.SemaphoreType.DMA,
            pltpu.SemaphoreType.DMA,
            pltpu.SemaphoreType.DMA,
            pltpu.SemaphoreType.DMA,
        ],
    )
    def k(x_hbm, b_hbm, ones_hbm, zrow_hbm, zcnt_hbm, sums_hbm, cnts_hbm,
          xb0, xb1, ib0, ib1, ones, acc, accc,
          slx0, slx1, sli0, sli1, ssc0, ssc1):
        cid = lax.axis_index("c")
        sid = lax.axis_index("s")
        w = cid * NS + sid

        xbufs = (xb0, xb1)
        ibufs = (ib0, ib1)
        slx = (slx0, slx1)
        sli = (sli0, sli1)
        ssc = (ssc0, ssc1)

        def start_load(i):
            p = i % 2
            big = w + i * NW

            @pl.when(big < NBIG)
            def _():
                pltpu.async_copy(x_hbm.at[pl.ds(big * ROWS, ROWS), :],
                                 xbufs[p], slx[p])
                pltpu.async_copy(b_hbm.at[pl.ds(big * BIG, BIG), :],
                                 ibufs[p], sli[p])

        def wait_load(i):
            p = i % 2
            big = w + i * NW

            @pl.when(big < NBIG)
            def _():
                pltpu.make_async_copy(x_hbm.at[pl.ds(0, ROWS), :],
                                      xbufs[p], slx[p]).wait()
                pltpu.make_async_copy(b_hbm.at[pl.ds(0, BIG), :],
                                      ibufs[p], sli[p]).wait()

        def start_scat(i):
            p = i % 2
            big = w + i * NW

            @pl.when(big < NBIG)
            def _():
                for j in range(BIG):
                    pltpu.async_copy(xbufs[p].at[pl.ds(j * C, C)],
                                     acc.at[ibufs[p].at[j]], ssc[p], add=True)
                    pltpu.async_copy(ones, accc.at[ibufs[p].at[j]],
                                     ssc[p], add=True)

        def wait_scat(i):
            p = i % 2
            big = w + i * NW

            @pl.when(big < NBIG)
            def _():
                for j in range(BIG):
                    pltpu.make_async_copy(xbufs[p].at[pl.ds(j * C, C)],
                                          acc.at[ibufs[p].at[j]], ssc[p]).wait()
                    pltpu.make_async_copy(ones, accc.at[ibufs[p].at[j]],
                                          ssc[p]).wait()

        start_load(0)
        start_load(1)
        pltpu.sync_copy(ones_hbm.at[w], ones)

        r0 = sid * RPS

        @pl.when(sid < NS - 1)
        def _():
            pltpu.sync_copy(zrow_hbm.at[pl.ds(r0, RPS)], acc.at[pl.ds(r0, RPS)])
            pltpu.sync_copy(zcnt_hbm.at[pl.ds(r0, RPS)], accc.at[pl.ds(r0, RPS)])

        @pl.when(sid == NS - 1)
        def _():
            pltpu.sync_copy(zrow_hbm.at[pl.ds((NS - 1) * RPS, RPS_LAST)],
                            acc.at[pl.ds((NS - 1) * RPS, RPS_LAST)])
            pltpu.sync_copy(zcnt_hbm.at[pl.ds((NS - 1) * RPS, RPS_LAST)],
                            accc.at[pl.ds((NS - 1) * RPS, RPS_LAST)])

        plsc.subcore_barrier()

        for i in range(TRIPS):
            wait_load(i)
            start_scat(i)
            if i >= 1:
                wait_scat(i - 1)
                if i + 1 < TRIPS:
                    start_load(i + 1)
        wait_scat(TRIPS - 1)

        plsc.subcore_barrier()

        @pl.when(sid < NS - 1)
        def _():
            pltpu.sync_copy(acc.at[pl.ds(r0, RPS)],
                            sums_hbm.at[cid, pl.ds(r0, RPS), :])
            pltpu.sync_copy(accc.at[pl.ds(r0, RPS)],
                            cnts_hbm.at[cid, pl.ds(r0, RPS), :])

        @pl.when(sid == NS - 1)
        def _():
            pltpu.sync_copy(acc.at[pl.ds((NS - 1) * RPS, RPS_LAST)],
                            sums_hbm.at[cid, pl.ds((NS - 1) * RPS, RPS_LAST), :])
            pltpu.sync_copy(accc.at[pl.ds((NS - 1) * RPS, RPS_LAST)],
                            cnts_hbm.at[cid, pl.ds((NS - 1) * RPS, RPS_LAST), :])

    return k(x, batch2d, ones_c, zrow_c, zcnt_c)


def _tc_finish(psums, pcnts, W, b2d):
    def body(ps_ref, pc_ref, w_ref, b_ref, o_ref):
        sums = ps_ref[0] + ps_ref[1]
        cnt = pc_ref[0, :, 0:1] + pc_ref[1, :, 0:1]
        pooled = sums / jnp.maximum(cnt, 1.0)
        o_ref[...] = (
            jnp.dot(pooled, w_ref[...], preferred_element_type=jnp.float32)
            + b_ref[...]
        )

    return pl.pallas_call(
        body,
        out_shape=jax.ShapeDtypeStruct((G, D), jnp.float32),
    )(psums, pcnts, W, b2d)


def kernel(x, batch, W, b):
    ones_c = jnp.ones((NW, C, CNT_W), dtype=jnp.float32)
    zrow_c = jnp.zeros((G, D), dtype=jnp.float32)
    zcnt_c = jnp.zeros((G, CNT_W), dtype=jnp.float32)
    psums, pcnts = _sc_segment_sums(x, batch.reshape(NCHUNKS, C),
                                    ones_c, zrow_c, zcnt_c)
    return _tc_finish(psums, pcnts, W, b.reshape(1, D))


# speedup vs baseline: 6.9322x; 1.0219x over previous
import functools

import jax
import jax.numpy as jnp
from jax import lax
from jax.experimental import pallas as pl
from jax.experimental.pallas import tpu as pltpu
from jax.experimental.pallas import tpu_sc as plsc

N_NODES = 100000
D = 128
G = 1000

NC = 2
NS = 16
NW = NC * NS

C = 80
NCHUNKS = N_NODES // C
BIG = 5
ROWS = BIG * C
NBIG = N_NODES // ROWS
TRIPS = -(-NBIG // NW)

CNT_W = 16
RPS = 64
RPS_LAST = G - (NS - 1) * RPS


def _sc_segment_sums(x, batch2d, ones_c, zrow_c, zcnt_c):
    mesh = plsc.VectorSubcoreMesh(core_axis_name="c", subcore_axis_name="s")

    @functools.partial(
        pl.kernel,
        out_type=[
            jax.ShapeDtypeStruct((NC, G, D), jnp.float32),
            jax.ShapeDtypeStruct((NC, G, CNT_W), jnp.float32),
        ],
        mesh=mesh,
        compiler_params=pltpu.CompilerParams(use_tc_tiling_on_sc=False),
        scratch_types=[
            pltpu.VMEM((ROWS, D), jnp.float32),
            pltpu.VMEM((ROWS, D), jnp.float32),
            pltpu.VMEM((BIG, C), jnp.int32),
            pltpu.VMEM((BIG, C), jnp.int32),
            pltpu.VMEM((C, CNT_W), jnp.float32),
            pltpu.VMEM((RPS, D), jnp.float32),
            pltpu.VMEM((RPS, CNT_W), jnp.float32),
            pltpu.VMEM_SHARED((G, D), jnp.float32),
            pltpu.VMEM_SHARED((G, CNT_W), jnp.float32),
            pltpu.SemaphoreType.DMA,
            pltpu.SemaphoreType.DMA,
            pltpu.SemaphoreType.DMA,
            pltpu.SemaphoreType.DMA,
            pltpu.SemaphoreType.DMA,
            pltpu.SemaphoreType.DMA,
        ],
    )
    def k(x_hbm, b_hbm, ones_hbm, zrow_hbm, zcnt_hbm, sums_hbm, cnts_hbm,
          xb0, xb1, ib0, ib1, ones, zbuf, zcbuf, acc, accc,
          slx0, slx1, sli0, sli1, ssc0, ssc1):
        cid = lax.axis_index("c")
        sid = lax.axis_index("s")
        w = cid * NS + sid

        xbufs = (xb0, xb1)
        ibufs = (ib0, ib1)
        slx = (slx0, slx1)
        sli = (sli0, sli1)
        ssc = (ssc0, ssc1)

        def start_load(i):
            p = i % 2
            big = w + i * NW

            @pl.when(big < NBIG)
            def _():
                pltpu.async_copy(x_hbm.at[pl.ds(big * ROWS, ROWS), :],
                                 xbufs[p], slx[p])
                pltpu.async_copy(b_hbm.at[pl.ds(big * BIG, BIG), :],
                                 ibufs[p], sli[p])

        def wait_load(i):
            p = i % 2
            big = w + i * NW

            @pl.when(big < NBIG)
            def _():
                pltpu.make_async_copy(x_hbm.at[pl.ds(0, ROWS), :],
                                      xbufs[p], slx[p]).wait()
                pltpu.make_async_copy(b_hbm.at[pl.ds(0, BIG), :],
                                      ibufs[p], sli[p]).wait()

        def start_scat(i):
            p = i % 2
            big = w + i * NW

            @pl.when(big < NBIG)
            def _():
                for j in range(BIG):
                    pltpu.async_copy(xbufs[p].at[pl.ds(j * C, C)],
                                     acc.at[ibufs[p].at[j]], ssc[p], add=True)
                    pltpu.async_copy(ones, accc.at[ibufs[p].at[j]],
                                     ssc[p], add=True)

        def wait_scat(i):
            p = i % 2
            big = w + i * NW

            @pl.when(big < NBIG)
            def _():
                for j in range(BIG):
                    pltpu.make_async_copy(xbufs[p].at[pl.ds(j * C, C)],
                                          acc.at[ibufs[p].at[j]], ssc[p]).wait()
                    pltpu.make_async_copy(ones, accc.at[ibufs[p].at[j]],
                                          ssc[p]).wait()

        start_load(0)
        start_load(1)
        pltpu.sync_copy(ones_hbm.at[w], ones)

        r0 = sid * RPS
        pltpu.sync_copy(zrow_hbm.at[pl.ds(r0, RPS)], zbuf)
        pltpu.sync_copy(zcnt_hbm.at[pl.ds(r0, RPS)], zcbuf)

        @pl.when(sid < NS - 1)
        def _():
            pltpu.sync_copy(zbuf, acc.at[pl.ds(r0, RPS)])
            pltpu.sync_copy(zcbuf, accc.at[pl.ds(r0, RPS)])

        @pl.when(sid == NS - 1)
        def _():
            pltpu.sync_copy(zbuf.at[pl.ds(0, RPS_LAST)],
                            acc.at[pl.ds((NS - 1) * RPS, RPS_LAST)])
            pltpu.sync_copy(zcbuf.at[pl.ds(0, RPS_LAST)],
                            accc.at[pl.ds((NS - 1) * RPS, RPS_LAST)])

        plsc.subcore_barrier()

        for i in range(TRIPS):
            wait_load(i)
            start_scat(i)
            if i >= 1:
                wait_scat(i - 1)
                if i + 1 < TRIPS:
                    start_load(i + 1)
        wait_scat(TRIPS - 1)

        plsc.subcore_barrier()

        @pl.when(sid < NS - 1)
        def _():
            pltpu.sync_copy(acc.at[pl.ds(r0, RPS)], zbuf)
            pltpu.sync_copy(accc.at[pl.ds(r0, RPS)], zcbuf)
            pltpu.sync_copy(zbuf, sums_hbm.at[cid, pl.ds(r0, RPS), :])
            pltpu.sync_copy(zcbuf, cnts_hbm.at[cid, pl.ds(r0, RPS), :])

        @pl.when(sid == NS - 1)
        def _():
            pltpu.sync_copy(acc.at[pl.ds((NS - 1) * RPS, RPS_LAST)],
                            zbuf.at[pl.ds(0, RPS_LAST)])
            pltpu.sync_copy(accc.at[pl.ds((NS - 1) * RPS, RPS_LAST)],
                            zcbuf.at[pl.ds(0, RPS_LAST)])
            pltpu.sync_copy(zbuf.at[pl.ds(0, RPS_LAST)],
                            sums_hbm.at[cid, pl.ds((NS - 1) * RPS, RPS_LAST), :])
            pltpu.sync_copy(zcbuf.at[pl.ds(0, RPS_LAST)],
                            cnts_hbm.at[cid, pl.ds((NS - 1) * RPS, RPS_LAST), :])

    return k(x, batch2d, ones_c, zrow_c, zcnt_c)


def _tc_finish(psums, pcnts, W, b2d):
    def body(ps_ref, pc_ref, w_ref, b_ref, o_ref):
        sums = ps_ref[0] + ps_ref[1]
        cnt = pc_ref[0, :, 0:1] + pc_ref[1, :, 0:1]
        pooled = sums / jnp.maximum(cnt, 1.0)
        o_ref[...] = (
            jnp.dot(pooled, w_ref[...], preferred_element_type=jnp.float32)
            + b_ref[...]
        )

    return pl.pallas_call(
        body,
        out_shape=jax.ShapeDtypeStruct((G, D), jnp.float32),
    )(psums, pcnts, W, b2d)


def kernel(x, batch, W, b):
    ones_c = jnp.ones((NW, C, CNT_W), dtype=jnp.float32)
    zrow_c = jnp.zeros((G, D), dtype=jnp.float32)
    zcnt_c = jnp.zeros((G, CNT_W), dtype=jnp.float32)
    psums, pcnts = _sc_segment_sums(x, batch.reshape(NCHUNKS, C),
                                    ones_c, zrow_c, zcnt_c)
    return _tc_finish(psums, pcnts, W, b.reshape(1, D))
